# Initial kernel scaffold; baseline (speedup 1.0000x reference)
import jax
import jax.numpy as jnp
from jax.experimental import pallas as pl


def kernel(x, adj, W1, b1, W11, b11, W12, b12):
    raise NotImplementedError("write your pallas kernel here")


# speedup vs baseline: 1.2089x; 1.2089x over previous
import functools

import jax
import jax.numpy as jnp
from jax.experimental import pallas as pl


def _support_body(x_ref, w1_ref, out_ref):
    out_ref[...] = jnp.dot(x_ref[...], w1_ref[...],
                           preferred_element_type=jnp.float32)


def _layer1_body(adj_ref, sup_ref, b1_ref, wc_ref, out_ref):
    h = jnp.dot(adj_ref[...], sup_ref[...], preferred_element_type=jnp.float32)
    h = jnp.maximum(h + b1_ref[...], 0.0)
    out_ref[...] = jnp.dot(h, wc_ref[...], preferred_element_type=jnp.float32)


def _layer2_body(adj_ref, s2_ref, bc_ref, eps_ref, out_ref, *, nclass):
    acc = jnp.dot(adj_ref[...], s2_ref[...], preferred_element_type=jnp.float32)
    acc = acc + bc_ref[...]
    mean = acc[:, :nclass]
    logstd = acc[:, nclass:]
    z = eps_ref[...] * jnp.exp(logstd) + mean
    m = jnp.max(z, axis=1, keepdims=True)
    zs = z - m
    lse = jnp.log(jnp.sum(jnp.exp(zs), axis=1, keepdims=True))
    out_ref[...] = zs - lse


def kernel(x, adj, W1, b1, W11, b11, W12, b12):
    n, nfeat = x.shape
    nhid = W1.shape[1]
    nclass = W11.shape[1]

    bi = 400 if n % 400 == 0 else 8
    grid = (n // bi,)

    wc = jnp.concatenate([W11, W12], axis=1)
    bc = jnp.concatenate([b11, b12])[None, :]
    b1r = b1[None, :]
    eps = jax.random.normal(jax.random.key(42), (n, nclass), dtype=jnp.float32)

    support = pl.pallas_call(
        _support_body,
        grid=grid,
        in_specs=[
            pl.BlockSpec((bi, nfeat), lambda i: (i, 0)),
            pl.BlockSpec((nfeat, nhid), lambda i: (0, 0)),
        ],
        out_specs=pl.BlockSpec((bi, nhid), lambda i: (i, 0)),
        out_shape=jax.ShapeDtypeStruct((n, nhid), jnp.float32),
    )(x, W1)

    s2 = pl.pallas_call(
        _layer1_body,
        grid=grid,
        in_specs=[
            pl.BlockSpec((bi, n), lambda i: (i, 0)),
            pl.BlockSpec((n, nhid), lambda i: (0, 0)),
            pl.BlockSpec((1, nhid), lambda i: (0, 0)),
            pl.BlockSpec((nhid, 2 * nclass), lambda i: (0, 0)),
        ],
        out_specs=pl.BlockSpec((bi, 2 * nclass), lambda i: (i, 0)),
        out_shape=jax.ShapeDtypeStruct((n, 2 * nclass), jnp.float32),
    )(adj, support, b1r, wc)

    out = pl.pallas_call(
        functools.partial(_layer2_body, nclass=nclass),
        grid=grid,
        in_specs=[
            pl.BlockSpec((bi, n), lambda i: (i, 0)),
            pl.BlockSpec((n, 2 * nclass), lambda i: (0, 0)),
            pl.BlockSpec((1, 2 * nclass), lambda i: (0, 0)),
            pl.BlockSpec((bi, nclass), lambda i: (i, 0)),
        ],
        out_specs=pl.BlockSpec((bi, nclass), lambda i: (i, 0)),
        out_shape=jax.ShapeDtypeStruct((n, nclass), jnp.float32),
    )(adj, s2, bc, eps)

    return out
